# Initial kernel scaffold; baseline (speedup 1.0000x reference)
import jax
import jax.numpy as jnp
from jax.experimental import pallas as pl


def kernel(exer_list, score_list, exer_conc_adj, exer_conc_w, conc_conc_w, exer_pote_w, lambd, guess, slide):
    raise NotImplementedError("write your pallas kernel here")


# speedup vs baseline: 2.6078x; 2.6078x over previous
import functools

import jax
import jax.numpy as jnp
from jax import lax
from jax.experimental import pallas as pl
from jax.experimental.pallas import tpu as pltpu
from jax.experimental.pallas import tpu_sc as plsc


def _w_body(ecw_ref, adj_ref, w_ref):
    w_ref[...] = jax.nn.sigmoid(ecw_ref[...]) * adj_ref[...]


def _compute_w(ecw, adj):
    E, C = ecw.shape
    ET = 2000
    return pl.pallas_call(
        _w_body,
        grid=(E // ET,),
        in_specs=[pl.BlockSpec((ET, C), lambda i: (i, 0)),
                  pl.BlockSpec((ET, C), lambda i: (i, 0))],
        out_specs=pl.BlockSpec((ET, C), lambda i: (i, 0)),
        out_shape=jax.ShapeDtypeStruct((E, C), jnp.float32),
    )(ecw, adj)


_CH0 = 104
_CH1 = 96


def _sc_gather(w, el_flat, x_flat, pote, B, L):
    E, C = w.shape
    P = pote.shape[1]
    NC, NS = 2, 16
    NW = NC * NS
    SPW = B // NW
    mesh = plsc.VectorSubcoreMesh(core_axis_name="c", subcore_axis_name="s")

    @functools.partial(
        pl.kernel,
        mesh=mesh,
        compiler_params=pltpu.CompilerParams(use_tc_tiling_on_sc=False,
                                             needs_layout_passes=False),
        out_type=[jax.ShapeDtypeStruct((B, C), jnp.float32),
                  jax.ShapeDtypeStruct((B, C), jnp.float32),
                  jax.ShapeDtypeStruct((B * L, P), jnp.float32)],
        scratch_types=[
            pltpu.VMEM((_CH0, C), jnp.float32),
            pltpu.VMEM((_CH1, C), jnp.float32),
            pltpu.VMEM((_CH0, P), jnp.float32),
            pltpu.VMEM((_CH1, P), jnp.float32),
            pltpu.VMEM((_CH0,), jnp.int32),
            pltpu.VMEM((_CH1,), jnp.int32),
            pltpu.VMEM((L,), jnp.float32),
            pltpu.VMEM((C,), jnp.float32),
            pltpu.VMEM((C,), jnp.float32),
            pltpu.SemaphoreType.DMA,
        ],
    )
    def sc_k(w_hbm, el_hbm, x_hbm, pote_hbm, u_hbm, s_hbm, g_hbm,
             rows_a, rows_b, pote_a, pote_b, idx_a, idx_b, x_v, u_v, s_v, sem):
        wid = lax.axis_index("s") * NC + lax.axis_index("c")

        def student_body(t, carry):
            i = wid * SPW + t
            base = pl.multiple_of(i * L, 8)
            pltpu.sync_copy(el_hbm.at[pl.ds(base, _CH0)], idx_a)
            pltpu.sync_copy(el_hbm.at[pl.ds(base + _CH0, _CH1)], idx_b)
            pltpu.sync_copy(x_hbm.at[pl.ds(base, L)], x_v)
            pltpu.async_copy(w_hbm.at[idx_a], rows_a, sem).wait()
            pltpu.async_copy(w_hbm.at[idx_b], rows_b, sem).wait()
            pltpu.async_copy(pote_hbm.at[idx_a], pote_a, sem).wait()
            pltpu.async_copy(pote_hbm.at[idx_b], pote_b, sem).wait()
            pltpu.sync_copy(pote_a, g_hbm.at[pl.ds(base, _CH0)])
            pltpu.sync_copy(pote_b, g_hbm.at[pl.ds(base + _CH0, _CH1)])

            for cg in range(4):
                col0 = cg * 128

                def body_a(l, accs):
                    xb = plsc.load_gather(x_v, [jnp.full((16,), l, jnp.int32)])
                    new = []
                    for j in range(8):
                        r = rows_a[l, pl.ds(col0 + j * 16, 16)]
                        new.append(accs[j] + xb * r)
                        new.append(accs[8 + j] + r)
                    return tuple(new[0::2]) + tuple(new[1::2])

                def body_b(l, accs):
                    xb = plsc.load_gather(
                        x_v, [jnp.full((16,), _CH0 + l, jnp.int32)])
                    new = []
                    for j in range(8):
                        r = rows_b[l, pl.ds(col0 + j * 16, 16)]
                        new.append(accs[j] + xb * r)
                        new.append(accs[8 + j] + r)
                    return tuple(new[0::2]) + tuple(new[1::2])

                accs = tuple(jnp.zeros((16,), jnp.float32) for _ in range(16))
                accs = lax.fori_loop(0, _CH0, body_a, accs)
                accs = lax.fori_loop(0, _CH1, body_b, accs)
                for j in range(8):
                    u_v[pl.ds(col0 + j * 16, 16)] = accs[j]
                    s_v[pl.ds(col0 + j * 16, 16)] = accs[8 + j]

            pltpu.sync_copy(u_v, u_hbm.at[i])
            pltpu.sync_copy(s_v, s_hbm.at[i])
            return carry

        lax.fori_loop(0, SPW, student_body, 0)

    return sc_k(w, el_flat, x_flat, pote)


def _stu_body(u_ref, s_ref, g_ref, x_ref, cc_ref, a_ref, bm_ref):
    u = u_ref[...]
    s = s_ref[...]
    mask = s != 0.0
    maskf = mask.astype(jnp.float32)
    a1 = u * maskf / jnp.where(mask, s, 1.0)
    ew = jnp.exp(cc_ref[...])
    num = jnp.dot(a1, ew, preferred_element_type=jnp.float32)
    den = jnp.dot(maskf, ew, preferred_element_type=jnp.float32)
    a_ref[...] = num / den

    g = g_ref[...]
    m = jnp.max(g, axis=1, keepdims=True)
    z = jnp.exp(g - m)
    sz = jnp.sum(z, axis=1)
    wz = jnp.sum(x_ref[...][:, :, None] * z, axis=1)
    bm_ref[...] = wz / sz


def _stage_students(u, s, g3, x, cc):
    B, C = u.shape
    _, L, P = g3.shape
    TB = 128
    return pl.pallas_call(
        _stu_body,
        grid=(B // TB,),
        in_specs=[pl.BlockSpec((TB, C), lambda i: (i, 0)),
                  pl.BlockSpec((TB, C), lambda i: (i, 0)),
                  pl.BlockSpec((TB, L, P), lambda i: (i, 0, 0)),
                  pl.BlockSpec((TB, L), lambda i: (i, 0)),
                  pl.BlockSpec((C, C), lambda i: (0, 0))],
        out_specs=[pl.BlockSpec((TB, C), lambda i: (i, 0)),
                   pl.BlockSpec((TB, P), lambda i: (i, 0))],
        out_shape=[jax.ShapeDtypeStruct((B, C), jnp.float32),
                   jax.ShapeDtypeStruct((B, P), jnp.float32)],
    )(u, s, g3, x, cc)


def _epi_body(a_ref, bm_ref, w_ref, pote_ref, lam_ref, gu_ref, sl_ref, y_ref):
    w = w_ref[...]
    w2 = w / jnp.sum(w, axis=1, keepdims=True)
    ya = lax.dot_general(a_ref[...], w2, (((1,), (1,)), ((), ())),
                         preferred_element_type=jnp.float32)
    p = pote_ref[...]
    pz = jnp.exp(p - jnp.max(p, axis=1, keepdims=True))
    d2 = pz / jnp.sum(pz, axis=1, keepdims=True)
    yb = lax.dot_general(bm_ref[...], d2, (((1,), (1,)), ((), ())),
                         preferred_element_type=jnp.float32)
    lam = jax.nn.sigmoid(lam_ref[...])
    sl = jax.nn.sigmoid(sl_ref[...])
    gu = jax.nn.sigmoid(gu_ref[...])
    ymid = jnp.clip((1.0 - lam) * ya + lam * yb, 1e-8, 1.0 - 1e-8)
    y_ref[...] = (1.0 - sl) * ymid + gu * (1.0 - ymid)


def _stage_epilogue(a, bm, w, pote, lambd, guess, slide):
    B, C = a.shape
    E, P = pote.shape
    ET = 512
    grid = (E + ET - 1) // ET
    return pl.pallas_call(
        _epi_body,
        grid=(grid,),
        in_specs=[pl.BlockSpec((B, C), lambda i: (0, 0)),
                  pl.BlockSpec((B, P), lambda i: (0, 0)),
                  pl.BlockSpec((ET, C), lambda i: (i, 0)),
                  pl.BlockSpec((ET, P), lambda i: (i, 0)),
                  pl.BlockSpec((1, ET), lambda i: (0, i)),
                  pl.BlockSpec((1, ET), lambda i: (0, i)),
                  pl.BlockSpec((1, ET), lambda i: (0, i))],
        out_specs=pl.BlockSpec((B, ET), lambda i: (0, i)),
        out_shape=jax.ShapeDtypeStruct((B, E), jnp.float32),
    )(a, bm, w, pote, lambd, guess, slide)


def kernel(exer_list, score_list, exer_conc_adj, exer_conc_w, conc_conc_w,
           exer_pote_w, lambd, guess, slide):
    B, L = exer_list.shape
    E, C = exer_conc_w.shape
    P = exer_pote_w.shape[1]

    w = _compute_w(exer_conc_w, exer_conc_adj)
    el_flat = exer_list.reshape(B * L).astype(jnp.int32)
    x_flat = score_list.reshape(B * L)
    u, s, g = _sc_gather(w, el_flat, x_flat, exer_pote_w, B, L)
    a, bm = _stage_students(u, s, g.reshape(B, L, P), score_list, conc_conc_w)
    y = _stage_epilogue(a, bm, w, exer_pote_w, lambd, guess, slide)
    return a, y


# speedup vs baseline: 3.4521x; 1.3238x over previous
import functools

import jax
import jax.numpy as jnp
from jax import lax
from jax.experimental import pallas as pl
from jax.experimental.pallas import tpu as pltpu
from jax.experimental.pallas import tpu_sc as plsc


def _w_body(ecw_ref, adj_ref, w_ref, wp_ref):
    w = jax.nn.sigmoid(ecw_ref[...]) * adj_ref[...]
    w_ref[...] = w
    wp_ref[...] = w.astype(jnp.bfloat16)


def _compute_w(ecw, adj):
    E, C = ecw.shape
    ET = 2000
    return pl.pallas_call(
        _w_body,
        grid=(E // ET,),
        in_specs=[pl.BlockSpec((ET, C), lambda i: (i, 0)),
                  pl.BlockSpec((ET, C), lambda i: (i, 0))],
        out_specs=[pl.BlockSpec((ET, C), lambda i: (i, 0)),
                   pl.BlockSpec((ET, C), lambda i: (i, 0))],
        out_shape=[jax.ShapeDtypeStruct((E, C), jnp.float32),
                   jax.ShapeDtypeStruct((E, C), jnp.bfloat16)],
    )(ecw, adj)


_CH0 = 104
_CH1 = 96


def _sc_gather(wp, el_flat, x_flat, pote, B, L):
    E, C = wp.shape
    P = pote.shape[1]
    NC, NS = 2, 16
    NW = NC * NS
    SPW = B // NW
    NCH = C // 32
    mesh = plsc.VectorSubcoreMesh(core_axis_name="c", subcore_axis_name="s")

    @functools.partial(
        pl.kernel,
        mesh=mesh,
        compiler_params=pltpu.CompilerParams(use_tc_tiling_on_sc=False,
                                             needs_layout_passes=False),
        out_type=[jax.ShapeDtypeStruct((B, C), jnp.float32),
                  jax.ShapeDtypeStruct((B, C), jnp.float32),
                  jax.ShapeDtypeStruct((B * L, P), jnp.float32)],
        scratch_types=[
            pltpu.VMEM((_CH0, C), jnp.bfloat16),
            pltpu.VMEM((_CH1, C), jnp.bfloat16),
            pltpu.VMEM((2, _CH0, P), jnp.float32),
            pltpu.VMEM((2, _CH1, P), jnp.float32),
            pltpu.VMEM((2, _CH0), jnp.int32),
            pltpu.VMEM((2, _CH1), jnp.int32),
            pltpu.VMEM((2, L), jnp.float32),
            pltpu.VMEM((2, C), jnp.float32),
            pltpu.VMEM((2, C), jnp.float32),
            pltpu.SemaphoreType.DMA,
            pltpu.SemaphoreType.DMA,
            pltpu.SemaphoreType.DMA,
            pltpu.SemaphoreType.DMA,
            pltpu.SemaphoreType.DMA,
            pltpu.SemaphoreType.DMA,
        ],
    )
    def sc_k(wp_hbm, el_hbm, x_hbm, pote_hbm, u_hbm, s_hbm, g_hbm,
             rows_a, rows_b, pote_a, pote_b, idx_a, idx_b, x_v, u_v, s_v,
             sem_a, sem_b, sem_ix, sem_pg, sem_gw, sem_us):
        wid = lax.axis_index("s") * NC + lax.axis_index("c")

        def sbase(t):
            return pl.multiple_of((wid * SPW + t) * L, 8)

        def accumulate(rows, nrows, loff, p, accs, cg):
            def body(l, a):
                xb = plsc.load_gather(
                    x_v, [jnp.full((16,), p, jnp.int32),
                          jnp.full((16,), loff + l, jnp.int32)])
                new = list(a)
                for j in range(4):
                    ck = cg * 4 + j
                    r = rows[l, pl.ds(ck * 32, 32)]
                    e, o = plsc.unpack(r, format=plsc.PackFormat.INTERLEAVED)
                    new[4 * j + 0] = new[4 * j + 0] + xb * e
                    new[4 * j + 1] = new[4 * j + 1] + xb * o
                    new[4 * j + 2] = new[4 * j + 2] + e
                    new[4 * j + 3] = new[4 * j + 3] + o
                return tuple(new)
            return lax.fori_loop(0, nrows, body, accs)

        def student_body(t, carry):
            p = lax.rem(t, 2)
            np_ = 1 - p
            base = sbase(t)
            i = wid * SPW + t

            pltpu.async_copy(pote_a.at[p], g_hbm.at[pl.ds(base, _CH0)], sem_gw)
            pltpu.async_copy(pote_b.at[p],
                             g_hbm.at[pl.ds(base + _CH0, _CH1)], sem_gw)

            @pl.when(t < SPW - 1)
            def _():
                nbase = sbase(t + 1)
                pltpu.async_copy(el_hbm.at[pl.ds(nbase, _CH0)],
                                 idx_a.at[np_], sem_ix)
                pltpu.async_copy(el_hbm.at[pl.ds(nbase + _CH0, _CH1)],
                                 idx_b.at[np_], sem_ix)
                pltpu.async_copy(x_hbm.at[pl.ds(nbase, L)], x_v.at[np_], sem_ix)

            pltpu.async_copy(wp_hbm.at[idx_b.at[p]], rows_b, sem_b)
            pltpu.make_async_copy(wp_hbm.at[idx_a.at[p]], rows_a, sem_a).wait()

            accs = [tuple(jnp.zeros((16,), jnp.float32) for _ in range(16))
                    for _ in range(NCH // 4)]
            for cg in range(NCH // 4):
                accs[cg] = accumulate(rows_a, _CH0, 0, p, accs[cg], cg)

            @pl.when(t > 0)
            def _():
                pltpu.make_async_copy(
                    pote_a.at[np_], g_hbm.at[pl.ds(base, _CH0)], sem_gw).wait()
                pltpu.make_async_copy(
                    pote_b.at[np_], g_hbm.at[pl.ds(base, _CH1)], sem_gw).wait()

            @pl.when(t < SPW - 1)
            def _():
                for _c in (pltpu.make_async_copy(
                        el_hbm.at[pl.ds(base, _CH0)], idx_a.at[np_], sem_ix),
                           pltpu.make_async_copy(
                        el_hbm.at[pl.ds(base, _CH1)], idx_b.at[np_], sem_ix),
                           pltpu.make_async_copy(
                        x_hbm.at[pl.ds(base, L)], x_v.at[np_], sem_ix)):
                    _c.wait()
                pltpu.async_copy(pote_hbm.at[idx_a.at[np_]],
                                 pote_a.at[np_], sem_pg)
                pltpu.async_copy(pote_hbm.at[idx_b.at[np_]],
                                 pote_b.at[np_], sem_pg)
                pltpu.async_copy(wp_hbm.at[idx_a.at[np_]], rows_a, sem_a)

            pltpu.make_async_copy(wp_hbm.at[idx_b.at[p]], rows_b, sem_b).wait()
            for cg in range(NCH // 4):
                accs[cg] = accumulate(rows_b, _CH1, _CH0, p, accs[cg], cg)

            pvec = jnp.full((16,), p, jnp.int32)
            ioe = 2 * lax.iota(jnp.int32, 16)
            for cg in range(NCH // 4):
                for j in range(4):
                    ck = cg * 4 + j
                    plsc.store_scatter(u_v, [pvec, ck * 32 + ioe],
                                       accs[cg][4 * j + 0])
                    plsc.store_scatter(u_v, [pvec, ck * 32 + 1 + ioe],
                                       accs[cg][4 * j + 1])
                    plsc.store_scatter(s_v, [pvec, ck * 32 + ioe],
                                       accs[cg][4 * j + 2])
                    plsc.store_scatter(s_v, [pvec, ck * 32 + 1 + ioe],
                                       accs[cg][4 * j + 3])

            @pl.when(t > 0)
            def _():
                pltpu.make_async_copy(u_v.at[np_], u_hbm.at[i], sem_us).wait()
                pltpu.make_async_copy(s_v.at[np_], s_hbm.at[i], sem_us).wait()

            pltpu.async_copy(u_v.at[p], u_hbm.at[i], sem_us)
            pltpu.async_copy(s_v.at[p], s_hbm.at[i], sem_us)

            @pl.when(t < SPW - 1)
            def _():
                pltpu.make_async_copy(pote_hbm.at[idx_a.at[np_]],
                                      pote_a.at[np_], sem_pg).wait()
                pltpu.make_async_copy(pote_hbm.at[idx_b.at[np_]],
                                      pote_b.at[np_], sem_pg).wait()

            return carry

        base0 = sbase(0)
        pltpu.sync_copy(el_hbm.at[pl.ds(base0, _CH0)], idx_a.at[0])
        pltpu.sync_copy(el_hbm.at[pl.ds(base0 + _CH0, _CH1)], idx_b.at[0])
        pltpu.sync_copy(x_hbm.at[pl.ds(base0, L)], x_v.at[0])
        pltpu.async_copy(pote_hbm.at[idx_a.at[0]], pote_a.at[0], sem_pg)
        pltpu.async_copy(pote_hbm.at[idx_b.at[0]], pote_b.at[0], sem_pg)
        pltpu.async_copy(wp_hbm.at[idx_a.at[0]], rows_a, sem_a)
        pltpu.make_async_copy(pote_hbm.at[idx_a.at[0]],
                              pote_a.at[0], sem_pg).wait()
        pltpu.make_async_copy(pote_hbm.at[idx_b.at[0]],
                              pote_b.at[0], sem_pg).wait()

        lax.fori_loop(0, SPW, student_body, 0)

        lastp = (SPW - 1) % 2
        lbase = sbase(SPW - 1)
        li = wid * SPW + SPW - 1
        pltpu.make_async_copy(pote_a.at[lastp],
                              g_hbm.at[pl.ds(lbase, _CH0)], sem_gw).wait()
        pltpu.make_async_copy(pote_b.at[lastp],
                              g_hbm.at[pl.ds(lbase, _CH1)], sem_gw).wait()
        pltpu.make_async_copy(u_v.at[lastp], u_hbm.at[li], sem_us).wait()
        pltpu.make_async_copy(s_v.at[lastp], s_hbm.at[li], sem_us).wait()

    return sc_k(wp, el_flat, x_flat, pote)


def _stu_body(u_ref, s_ref, g_ref, x_ref, cc_ref, a_ref, bm_ref):
    u = u_ref[...]
    s = s_ref[...]
    mask = s != 0.0
    maskf = mask.astype(jnp.float32)
    a1 = u * maskf / jnp.where(mask, s, 1.0)
    ew = jnp.exp(cc_ref[...])
    num = jnp.dot(a1, ew, preferred_element_type=jnp.float32)
    den = jnp.dot(maskf, ew, preferred_element_type=jnp.float32)
    a_ref[...] = num / den

    g = g_ref[...]
    m = jnp.max(g, axis=1, keepdims=True)
    z = jnp.exp(g - m)
    sz = jnp.sum(z, axis=1)
    wz = jnp.sum(x_ref[...][:, :, None] * z, axis=1)
    bm_ref[...] = wz / sz


def _stage_students(u, s, g3, x, cc):
    B, C = u.shape
    _, L, P = g3.shape
    TB = 128
    return pl.pallas_call(
        _stu_body,
        grid=(B // TB,),
        in_specs=[pl.BlockSpec((TB, C), lambda i: (i, 0)),
                  pl.BlockSpec((TB, C), lambda i: (i, 0)),
                  pl.BlockSpec((TB, L, P), lambda i: (i, 0, 0)),
                  pl.BlockSpec((TB, L), lambda i: (i, 0)),
                  pl.BlockSpec((C, C), lambda i: (0, 0))],
        out_specs=[pl.BlockSpec((TB, C), lambda i: (i, 0)),
                   pl.BlockSpec((TB, P), lambda i: (i, 0))],
        out_shape=[jax.ShapeDtypeStruct((B, C), jnp.float32),
                   jax.ShapeDtypeStruct((B, P), jnp.float32)],
    )(u, s, g3, x, cc)


def _epi_body(a_ref, bm_ref, w_ref, pote_ref, lam_ref, gu_ref, sl_ref, y_ref):
    w = w_ref[...]
    w2 = w / jnp.sum(w, axis=1, keepdims=True)
    ya = lax.dot_general(a_ref[...], w2, (((1,), (1,)), ((), ())),
                         preferred_element_type=jnp.float32)
    p = pote_ref[...]
    pz = jnp.exp(p - jnp.max(p, axis=1, keepdims=True))
    d2 = pz / jnp.sum(pz, axis=1, keepdims=True)
    yb = lax.dot_general(bm_ref[...], d2, (((1,), (1,)), ((), ())),
                         preferred_element_type=jnp.float32)
    lam = jax.nn.sigmoid(lam_ref[...])
    sl = jax.nn.sigmoid(sl_ref[...])
    gu = jax.nn.sigmoid(gu_ref[...])
    ymid = jnp.clip((1.0 - lam) * ya + lam * yb, 1e-8, 1.0 - 1e-8)
    y_ref[...] = (1.0 - sl) * ymid + gu * (1.0 - ymid)


def _stage_epilogue(a, bm, w, pote, lambd, guess, slide):
    B, C = a.shape
    E, P = pote.shape
    ET = 512
    grid = (E + ET - 1) // ET
    return pl.pallas_call(
        _epi_body,
        grid=(grid,),
        in_specs=[pl.BlockSpec((B, C), lambda i: (0, 0)),
                  pl.BlockSpec((B, P), lambda i: (0, 0)),
                  pl.BlockSpec((ET, C), lambda i: (i, 0)),
                  pl.BlockSpec((ET, P), lambda i: (i, 0)),
                  pl.BlockSpec((1, ET), lambda i: (0, i)),
                  pl.BlockSpec((1, ET), lambda i: (0, i)),
                  pl.BlockSpec((1, ET), lambda i: (0, i))],
        out_specs=pl.BlockSpec((B, ET), lambda i: (0, i)),
        out_shape=jax.ShapeDtypeStruct((B, E), jnp.float32),
    )(a, bm, w, pote, lambd, guess, slide)


def kernel(exer_list, score_list, exer_conc_adj, exer_conc_w, conc_conc_w,
           exer_pote_w, lambd, guess, slide):
    B, L = exer_list.shape
    E, C = exer_conc_w.shape
    P = exer_pote_w.shape[1]

    w, wp = _compute_w(exer_conc_w, exer_conc_adj)
    el_flat = exer_list.reshape(B * L).astype(jnp.int32)
    x_flat = score_list.reshape(B * L)
    u, s, g = _sc_gather(wp, el_flat, x_flat, exer_pote_w, B, L)
    a, bm = _stage_students(u, s, g.reshape(B, L, P), score_list, conc_conc_w)
    y = _stage_epilogue(a, bm, w, exer_pote_w, lambd, guess, slide)
    return a, y


# speedup vs baseline: 4.4279x; 1.2827x over previous
import functools

import jax
import jax.numpy as jnp
from jax import lax
from jax.experimental import pallas as pl
from jax.experimental.pallas import tpu as pltpu
from jax.experimental.pallas import tpu_sc as plsc


def _w_body(ecw_ref, adj_ref, wp_ref):
    w = jax.nn.sigmoid(ecw_ref[...]) * adj_ref[...]
    wp_ref[...] = w.astype(jnp.bfloat16)


def _compute_w(ecw, adj):
    E, C = ecw.shape
    ET = 2000
    return pl.pallas_call(
        _w_body,
        grid=(E // ET,),
        in_specs=[pl.BlockSpec((ET, C), lambda i: (i, 0)),
                  pl.BlockSpec((ET, C), lambda i: (i, 0))],
        out_specs=pl.BlockSpec((ET, C), lambda i: (i, 0)),
        out_shape=jax.ShapeDtypeStruct((E, C), jnp.bfloat16),
    )(ecw, adj)


_CH0 = 104
_CH1 = 96


def _sc_gather(wp, el, x, pote, B, L):
    E, C = wp.shape
    P = pote.shape[1]
    NC, NS = 2, 16
    NW = NC * NS
    SPW = B // NW
    NCH = C // 32
    mesh = plsc.VectorSubcoreMesh(core_axis_name="c", subcore_axis_name="s")

    @functools.partial(
        pl.kernel,
        mesh=mesh,
        compiler_params=pltpu.CompilerParams(use_tc_tiling_on_sc=False,
                                             needs_layout_passes=False),
        out_type=[jax.ShapeDtypeStruct((B, C), jnp.float32),
                  jax.ShapeDtypeStruct((B, C), jnp.float32),
                  jax.ShapeDtypeStruct((B * L, P), jnp.float32)],
        scratch_types=[
            pltpu.VMEM((_CH0, C), jnp.bfloat16),
            pltpu.VMEM((_CH1, C), jnp.bfloat16),
            pltpu.VMEM((2, _CH0, P), jnp.float32),
            pltpu.VMEM((2, _CH1, P), jnp.float32),
            pltpu.VMEM((2, _CH0), jnp.int32),
            pltpu.VMEM((2, _CH1), jnp.int32),
            pltpu.VMEM((2, L), jnp.float32),
            pltpu.VMEM((2, C), jnp.float32),
            pltpu.VMEM((2, C), jnp.float32),
            pltpu.SemaphoreType.DMA,
            pltpu.SemaphoreType.DMA,
            pltpu.SemaphoreType.DMA,
            pltpu.SemaphoreType.DMA,
            pltpu.SemaphoreType.DMA,
            pltpu.SemaphoreType.DMA,
        ],
    )
    def sc_k(wp_hbm, el_hbm, x_hbm, pote_hbm, u_hbm, s_hbm, g_hbm,
             rows_a, rows_b, pote_a, pote_b, idx_a, idx_b, x_v, u_v, s_v,
             sem_a, sem_b, sem_ix, sem_pg, sem_gw, sem_us):
        wid = lax.axis_index("s") * NC + lax.axis_index("c")

        def sbase(t):
            return pl.multiple_of((wid * SPW + t) * L, 8)

        def accumulate(rows, nrows, loff, p, accs, cg):
            def body(l, a):
                xb = plsc.load_gather(
                    x_v, [jnp.full((16,), p, jnp.int32),
                          jnp.full((16,), loff + l, jnp.int32)])
                new = list(a)
                for j in range(4):
                    ck = cg * 4 + j
                    r = rows[l, pl.ds(ck * 32, 32)]
                    e, o = plsc.unpack(r, format=plsc.PackFormat.INTERLEAVED)
                    new[4 * j + 0] = new[4 * j + 0] + xb * e
                    new[4 * j + 1] = new[4 * j + 1] + xb * o
                    new[4 * j + 2] = new[4 * j + 2] + e
                    new[4 * j + 3] = new[4 * j + 3] + o
                return tuple(new)
            return plsc.parallel_loop(0, nrows, unroll=4, carry=accs)(body)

        def student_body(t, carry):
            p = lax.rem(t, 2)
            np_ = 1 - p
            base = sbase(t)
            i = wid * SPW + t

            pltpu.async_copy(pote_a.at[p], g_hbm.at[pl.ds(base, _CH0)], sem_gw)
            pltpu.async_copy(pote_b.at[p],
                             g_hbm.at[pl.ds(base + _CH0, _CH1)], sem_gw)

            @pl.when(t < SPW - 1)
            def _():
                ni = wid * SPW + t + 1
                pltpu.async_copy(el_hbm.at[ni, pl.ds(0, _CH0)],
                                 idx_a.at[np_], sem_ix)
                pltpu.async_copy(el_hbm.at[ni, pl.ds(_CH0, _CH1)],
                                 idx_b.at[np_], sem_ix)
                pltpu.async_copy(x_hbm.at[ni], x_v.at[np_], sem_ix)

            pltpu.async_copy(wp_hbm.at[idx_b.at[p]], rows_b, sem_b)
            pltpu.make_async_copy(wp_hbm.at[idx_a.at[p]], rows_a, sem_a).wait()

            accs = [tuple(jnp.zeros((16,), jnp.float32) for _ in range(16))
                    for _ in range(NCH // 4)]
            for cg in range(NCH // 4):
                accs[cg] = accumulate(rows_a, _CH0, 0, p, accs[cg], cg)

            @pl.when(t > 0)
            def _():
                pltpu.make_async_copy(
                    pote_a.at[np_], g_hbm.at[pl.ds(base, _CH0)], sem_gw).wait()
                pltpu.make_async_copy(
                    pote_b.at[np_], g_hbm.at[pl.ds(base, _CH1)], sem_gw).wait()

            @pl.when(t < SPW - 1)
            def _():
                for _c in (pltpu.make_async_copy(
                        el_hbm.at[i, pl.ds(0, _CH0)], idx_a.at[np_], sem_ix),
                           pltpu.make_async_copy(
                        el_hbm.at[i, pl.ds(_CH0, _CH1)], idx_b.at[np_], sem_ix),
                           pltpu.make_async_copy(
                        x_hbm.at[i], x_v.at[np_], sem_ix)):
                    _c.wait()
                pltpu.async_copy(pote_hbm.at[idx_a.at[np_]],
                                 pote_a.at[np_], sem_pg)
                pltpu.async_copy(pote_hbm.at[idx_b.at[np_]],
                                 pote_b.at[np_], sem_pg)
                pltpu.async_copy(wp_hbm.at[idx_a.at[np_]], rows_a, sem_a)

            pltpu.make_async_copy(wp_hbm.at[idx_b.at[p]], rows_b, sem_b).wait()
            for cg in range(NCH // 4):
                accs[cg] = accumulate(rows_b, _CH1, _CH0, p, accs[cg], cg)

            pvec = jnp.full((16,), p, jnp.int32)
            ioe = 2 * lax.iota(jnp.int32, 16)
            for cg in range(NCH // 4):
                for j in range(4):
                    ck = cg * 4 + j
                    plsc.store_scatter(u_v, [pvec, ck * 32 + ioe],
                                       accs[cg][4 * j + 0])
                    plsc.store_scatter(u_v, [pvec, ck * 32 + 1 + ioe],
                                       accs[cg][4 * j + 1])
                    plsc.store_scatter(s_v, [pvec, ck * 32 + ioe],
                                       accs[cg][4 * j + 2])
                    plsc.store_scatter(s_v, [pvec, ck * 32 + 1 + ioe],
                                       accs[cg][4 * j + 3])

            @pl.when(t > 0)
            def _():
                pltpu.make_async_copy(u_v.at[np_], u_hbm.at[i], sem_us).wait()
                pltpu.make_async_copy(s_v.at[np_], s_hbm.at[i], sem_us).wait()

            pltpu.async_copy(u_v.at[p], u_hbm.at[i], sem_us)
            pltpu.async_copy(s_v.at[p], s_hbm.at[i], sem_us)

            @pl.when(t < SPW - 1)
            def _():
                pltpu.make_async_copy(pote_hbm.at[idx_a.at[np_]],
                                      pote_a.at[np_], sem_pg).wait()
                pltpu.make_async_copy(pote_hbm.at[idx_b.at[np_]],
                                      pote_b.at[np_], sem_pg).wait()

            return carry

        i0 = wid * SPW
        pltpu.sync_copy(el_hbm.at[i0, pl.ds(0, _CH0)], idx_a.at[0])
        pltpu.sync_copy(el_hbm.at[i0, pl.ds(_CH0, _CH1)], idx_b.at[0])
        pltpu.sync_copy(x_hbm.at[i0], x_v.at[0])
        pltpu.async_copy(pote_hbm.at[idx_a.at[0]], pote_a.at[0], sem_pg)
        pltpu.async_copy(pote_hbm.at[idx_b.at[0]], pote_b.at[0], sem_pg)
        pltpu.async_copy(wp_hbm.at[idx_a.at[0]], rows_a, sem_a)
        pltpu.make_async_copy(pote_hbm.at[idx_a.at[0]],
                              pote_a.at[0], sem_pg).wait()
        pltpu.make_async_copy(pote_hbm.at[idx_b.at[0]],
                              pote_b.at[0], sem_pg).wait()

        lax.fori_loop(0, SPW, student_body, 0)

        lastp = (SPW - 1) % 2
        lbase = sbase(SPW - 1)
        li = wid * SPW + SPW - 1
        pltpu.make_async_copy(pote_a.at[lastp],
                              g_hbm.at[pl.ds(lbase, _CH0)], sem_gw).wait()
        pltpu.make_async_copy(pote_b.at[lastp],
                              g_hbm.at[pl.ds(lbase, _CH1)], sem_gw).wait()
        pltpu.make_async_copy(u_v.at[lastp], u_hbm.at[li], sem_us).wait()
        pltpu.make_async_copy(s_v.at[lastp], s_hbm.at[li], sem_us).wait()

    return sc_k(wp, el, x, pote)


def _make_stu_body(L, P):
    def _stu_body(u_ref, s_ref, g_ref, x0_ref, x1_ref, x2_ref, x3_ref,
                  cc_ref, a_ref, bm_ref):
        u = u_ref[...]
        s = s_ref[...]
        mask = s != 0.0
        maskf = mask.astype(jnp.float32)
        a1 = u * maskf / jnp.where(mask, s, 1.0)
        ew = jnp.exp(cc_ref[...])
        num = jnp.dot(a1, ew, preferred_element_type=jnp.float32)
        den = jnp.dot(maskf, ew, preferred_element_type=jnp.float32)
        a_ref[...] = num / den

        TB = u.shape[0]
        R = (L * P) // 128
        gz = g_ref[...].reshape(TB, R, 128)
        z = jnp.exp(gz)
        sz128 = jnp.sum(z, axis=1)
        sz = (sz128[:, 0:P] + sz128[:, P:2 * P]
              + sz128[:, 2 * P:3 * P] + sz128[:, 3 * P:])
        wz = (jnp.sum(x0_ref[...][:, :, None] * z[:, :, 0:P], axis=1)
              + jnp.sum(x1_ref[...][:, :, None] * z[:, :, P:2 * P], axis=1)
              + jnp.sum(x2_ref[...][:, :, None] * z[:, :, 2 * P:3 * P], axis=1)
              + jnp.sum(x3_ref[...][:, :, None] * z[:, :, 3 * P:], axis=1))
        bm_ref[...] = wz / sz
    return _stu_body


def _stage_students(u, s, g, x, cc, L, P):
    B, C = u.shape
    TB = 128
    R = (L * P) // 128
    g2 = g.reshape(B * R, 128)
    xq = [x[:, q::4] for q in range(4)]
    return pl.pallas_call(
        _make_stu_body(L, P),
        grid=(B // TB,),
        in_specs=[pl.BlockSpec((TB, C), lambda i: (i, 0)),
                  pl.BlockSpec((TB, C), lambda i: (i, 0)),
                  pl.BlockSpec((TB * R, 128), lambda i: (i, 0)),
                  pl.BlockSpec((TB, L // 4), lambda i: (i, 0)),
                  pl.BlockSpec((TB, L // 4), lambda i: (i, 0)),
                  pl.BlockSpec((TB, L // 4), lambda i: (i, 0)),
                  pl.BlockSpec((TB, L // 4), lambda i: (i, 0)),
                  pl.BlockSpec((C, C), lambda i: (0, 0))],
        out_specs=[pl.BlockSpec((TB, C), lambda i: (i, 0)),
                   pl.BlockSpec((TB, P), lambda i: (i, 0))],
        out_shape=[jax.ShapeDtypeStruct((B, C), jnp.float32),
                   jax.ShapeDtypeStruct((B, P), jnp.float32)],
    )(u, s, g2, xq[0], xq[1], xq[2], xq[3], cc)


def _epi_body(a_ref, bm_ref, w_ref, pote_ref, lam_ref, gu_ref, sl_ref, y_ref):
    w = w_ref[...].astype(jnp.float32)
    w2 = (w / jnp.sum(w, axis=1, keepdims=True)).astype(jnp.bfloat16)
    ya = lax.dot_general(w2, a_ref[...], (((1,), (1,)), ((), ())),
                         preferred_element_type=jnp.float32)
    p = pote_ref[...]
    pz = jnp.exp(p - jnp.max(p, axis=1, keepdims=True))
    d2 = (pz / jnp.sum(pz, axis=1, keepdims=True)).astype(jnp.bfloat16)
    yb = lax.dot_general(d2, bm_ref[...], (((1,), (1,)), ((), ())),
                         preferred_element_type=jnp.float32)
    ET = w.shape[0]
    lam = jax.nn.sigmoid(lam_ref[...]).reshape(ET, 1)
    sl = jax.nn.sigmoid(sl_ref[...]).reshape(ET, 1)
    gu = jax.nn.sigmoid(gu_ref[...]).reshape(ET, 1)
    ymid = jnp.clip((1.0 - lam) * ya + lam * yb, 1e-8, 1.0 - 1e-8)
    y_ref[...] = (1.0 - sl) * ymid + gu * (1.0 - ymid)


def _stage_epilogue(a, bm, w, pote, lambd, guess, slide):
    B, C = a.shape
    E, P = pote.shape
    ET = 1024
    grid = (E + ET - 1) // ET
    return pl.pallas_call(
        _epi_body,
        grid=(grid,),
        in_specs=[pl.BlockSpec((B, C), lambda i: (0, 0)),
                  pl.BlockSpec((B, P), lambda i: (0, 0)),
                  pl.BlockSpec((ET, C), lambda i: (i, 0)),
                  pl.BlockSpec((ET, P), lambda i: (i, 0)),
                  pl.BlockSpec((1, ET), lambda i: (0, i)),
                  pl.BlockSpec((1, ET), lambda i: (0, i)),
                  pl.BlockSpec((1, ET), lambda i: (0, i))],
        out_specs=pl.BlockSpec((ET, B), lambda i: (i, 0)),
        out_shape=jax.ShapeDtypeStruct((E, B), jnp.float32),
    )(a, bm, w, pote, lambd, guess, slide)


def kernel(exer_list, score_list, exer_conc_adj, exer_conc_w, conc_conc_w,
           exer_pote_w, lambd, guess, slide):
    B, L = exer_list.shape
    E, C = exer_conc_w.shape
    P = exer_pote_w.shape[1]

    wp = _compute_w(exer_conc_w, exer_conc_adj)
    u, s, g = _sc_gather(wp, exer_list, score_list, exer_pote_w, B, L)
    a, bm = _stage_students(u, s, g, score_list, conc_conc_w, L, P)
    yt = _stage_epilogue(a.astype(jnp.bfloat16), bm.astype(jnp.bfloat16),
                         wp, exer_pote_w, lambd, guess, slide)
    return a, yt.T
